# baseline (speedup 1.0000x reference)
import functools

import jax
import jax.numpy as jnp
from jax import lax
from jax.experimental import pallas as pl
from jax.experimental.pallas import tpu as pltpu
from jax.experimental.pallas import tpu_sc as plsc

BATCH = 16384
EMBED = 16
NUM_CORES = 2
NUM_SUBCORES = 16
NW = NUM_CORES * NUM_SUBCORES
B_PER_W = BATCH // NW
CHUNK = 128
N_CHUNKS = B_PER_W // CHUNK

MAX_RATING = 5.0
MIN_RATING = 0.5


U_BLK = 163840
M_BLK = 16384


def _matvec_body(ut_ref, mt_ref, wu_ref, wm_ref, ou_ref, om_ref):
    dn = (((1,), (0,)), ((), ()))
    ou = jax.lax.dot_general(wu_ref[...], ut_ref[...], dn,
                             preferred_element_type=jnp.float32)
    om = jax.lax.dot_general(wm_ref[...], mt_ref[...], dn,
                             preferred_element_type=jnp.float32)
    ou_ref[...] = ou.reshape(ou_ref.shape)
    om_ref[...] = om.reshape(om_ref.shape)


def _matvec2(ut, mt, wu_col, wm_col):
    nu = ut.shape[1]
    nm = mt.shape[1]
    grid = (nu + U_BLK - 1) // U_BLK
    assert (nm + M_BLK - 1) // M_BLK == grid
    return pl.pallas_call(
        _matvec_body,
        grid=(grid,),
        in_specs=[
            pl.BlockSpec((EMBED, U_BLK), lambda i: (0, i)),
            pl.BlockSpec((EMBED, M_BLK), lambda i: (0, i)),
            pl.BlockSpec((1, EMBED), lambda i: (0, 0)),
            pl.BlockSpec((1, EMBED), lambda i: (0, 0)),
        ],
        out_specs=(
            pl.BlockSpec((U_BLK,), lambda i: (i,)),
            pl.BlockSpec((M_BLK,), lambda i: (i,)),
        ),
        out_shape=(
            jax.ShapeDtypeStruct((nu,), jnp.float32),
            jax.ShapeDtypeStruct((nm,), jnp.float32),
        ),
        compiler_params=pltpu.CompilerParams(
            dimension_semantics=("parallel",),
        ),
    )(ut, mt, wu_col, wm_col)


def _sc_body(uid_hbm, mid_hbm, su_hbm, sm_hbm, b_hbm, out_hbm,
             uidx, midx, sus, sms, bvv, outv, usem, msem):
    wid = lax.axis_index("s") * NUM_CORES + lax.axis_index("c")
    base = wid * B_PER_W

    pltpu.sync_copy(uid_hbm.at[pl.ds(base, B_PER_W)], uidx)
    pltpu.sync_copy(mid_hbm.at[pl.ds(base, B_PER_W)], midx)
    pltpu.sync_copy(b_hbm, bvv)

    copies = []
    for c in range(N_CHUNKS):
        sl = pl.ds(c * CHUNK, CHUNK)
        copies.append(pltpu.async_copy(su_hbm.at[uidx.at[sl]], sus.at[sl], usem))
        copies.append(pltpu.async_copy(sm_hbm.at[midx.at[sl]], sms.at[sl], msem))

    bv = bvv[...]
    for cp in copies:
        cp.wait()

    def grp_body(g, carry):
        off = pl.multiple_of(g * EMBED, EMBED)
        v = sus[pl.ds(off, EMBED)] + sms[pl.ds(off, EMBED)] + bv
        y = (MAX_RATING - MIN_RATING) / (1.0 + jnp.exp(-v)) + MIN_RATING
        outv[pl.ds(off, EMBED)] = y
        return carry

    lax.fori_loop(0, B_PER_W // EMBED, grp_body, 0)

    pltpu.sync_copy(outv, out_hbm.at[pl.ds(base, B_PER_W)])


@functools.partial(
    pl.kernel,
    mesh=plsc.VectorSubcoreMesh(core_axis_name="c", subcore_axis_name="s"),
    out_type=jax.ShapeDtypeStruct((BATCH,), jnp.float32),
    compiler_params=pltpu.CompilerParams(
        needs_layout_passes=False, use_tc_tiling_on_sc=False
    ),
    scratch_types=[
        pltpu.VMEM((B_PER_W,), jnp.int32),
        pltpu.VMEM((B_PER_W,), jnp.int32),
        pltpu.VMEM((B_PER_W,), jnp.float32),
        pltpu.VMEM((B_PER_W,), jnp.float32),
        pltpu.VMEM((EMBED,), jnp.float32),
        pltpu.VMEM((B_PER_W,), jnp.float32),
        pltpu.SemaphoreType.DMA,
        pltpu.SemaphoreType.DMA,
    ],
)
def _sc_combine(*refs):
    _sc_body(*refs)


def kernel(user_ids, movie_ids, user_table, movie_table, W, b):
    uid = user_ids.astype(jnp.int32)
    mid = movie_ids.astype(jnp.int32)
    wflat = W.reshape(-1).astype(jnp.float32)
    wu_col = wflat[:EMBED].reshape(1, EMBED)
    wm_col = wflat[EMBED:].reshape(1, EMBED)
    bvec = jnp.broadcast_to(b.astype(jnp.float32), (EMBED,))
    su, sm = _matvec2(user_table.T, movie_table.T, wu_col, wm_col)
    return _sc_combine(uid, mid, su, sm, bvec)


# speedup vs baseline: 1.0267x; 1.0267x over previous
import functools

import jax
import jax.numpy as jnp
from jax import lax
from jax.experimental import pallas as pl
from jax.experimental.pallas import tpu as pltpu
from jax.experimental.pallas import tpu_sc as plsc

BATCH = 16384
EMBED = 16
NUM_CORES = 2
NUM_SUBCORES = 16
NW = NUM_CORES * NUM_SUBCORES
B_PER_W = BATCH // NW
CHUNK = 128
N_CHUNKS = B_PER_W // CHUNK

MAX_RATING = 5.0
MIN_RATING = 0.5


U_BLK = 102400
M_BLK = 10240


def _matvec_body(ut_ref, mt_ref, wu_ref, wm_ref, ou_ref, om_ref):
    dn = (((1,), (0,)), ((), ()))
    ou = jax.lax.dot_general(wu_ref[...], ut_ref[...], dn,
                             preferred_element_type=jnp.float32)
    om = jax.lax.dot_general(wm_ref[...], mt_ref[...], dn,
                             preferred_element_type=jnp.float32)
    ou_ref[...] = ou.reshape(ou_ref.shape)
    om_ref[...] = om.reshape(om_ref.shape)


def _matvec2(ut, mt, wu_col, wm_col):
    nu = ut.shape[1]
    nm = mt.shape[1]
    grid = (nu + U_BLK - 1) // U_BLK
    assert (nm + M_BLK - 1) // M_BLK == grid
    return pl.pallas_call(
        _matvec_body,
        grid=(grid,),
        in_specs=[
            pl.BlockSpec((EMBED, U_BLK), lambda i: (0, i)),
            pl.BlockSpec((EMBED, M_BLK), lambda i: (0, i)),
            pl.BlockSpec((1, EMBED), lambda i: (0, 0)),
            pl.BlockSpec((1, EMBED), lambda i: (0, 0)),
        ],
        out_specs=(
            pl.BlockSpec((U_BLK,), lambda i: (i,)),
            pl.BlockSpec((M_BLK,), lambda i: (i,)),
        ),
        out_shape=(
            jax.ShapeDtypeStruct((nu,), jnp.float32),
            jax.ShapeDtypeStruct((nm,), jnp.float32),
        ),
        compiler_params=pltpu.CompilerParams(
            dimension_semantics=("parallel",),
        ),
    )(ut, mt, wu_col, wm_col)


def _sc_body(uid_hbm, mid_hbm, su_hbm, sm_hbm, b_hbm, out_hbm,
             uidx, midx, sus, sms, bvv, outv, usem, msem):
    wid = lax.axis_index("s") * NUM_CORES + lax.axis_index("c")
    base = wid * B_PER_W

    pltpu.sync_copy(uid_hbm.at[pl.ds(base, B_PER_W)], uidx)
    pltpu.sync_copy(mid_hbm.at[pl.ds(base, B_PER_W)], midx)
    pltpu.sync_copy(b_hbm, bvv)

    copies = []
    for c in range(N_CHUNKS):
        sl = pl.ds(c * CHUNK, CHUNK)
        copies.append(pltpu.async_copy(su_hbm.at[uidx.at[sl]], sus.at[sl], usem))
        copies.append(pltpu.async_copy(sm_hbm.at[midx.at[sl]], sms.at[sl], msem))

    bv = bvv[...]
    for cp in copies:
        cp.wait()

    def grp_body(g, carry):
        off = pl.multiple_of(g * EMBED, EMBED)
        v = sus[pl.ds(off, EMBED)] + sms[pl.ds(off, EMBED)] + bv
        y = (MAX_RATING - MIN_RATING) / (1.0 + jnp.exp(-v)) + MIN_RATING
        outv[pl.ds(off, EMBED)] = y
        return carry

    lax.fori_loop(0, B_PER_W // EMBED, grp_body, 0)

    pltpu.sync_copy(outv, out_hbm.at[pl.ds(base, B_PER_W)])


@functools.partial(
    pl.kernel,
    mesh=plsc.VectorSubcoreMesh(core_axis_name="c", subcore_axis_name="s"),
    out_type=jax.ShapeDtypeStruct((BATCH,), jnp.float32),
    compiler_params=pltpu.CompilerParams(
        needs_layout_passes=False, use_tc_tiling_on_sc=False
    ),
    scratch_types=[
        pltpu.VMEM((B_PER_W,), jnp.int32),
        pltpu.VMEM((B_PER_W,), jnp.int32),
        pltpu.VMEM((B_PER_W,), jnp.float32),
        pltpu.VMEM((B_PER_W,), jnp.float32),
        pltpu.VMEM((EMBED,), jnp.float32),
        pltpu.VMEM((B_PER_W,), jnp.float32),
        pltpu.SemaphoreType.DMA,
        pltpu.SemaphoreType.DMA,
    ],
)
def _sc_combine(*refs):
    _sc_body(*refs)


def kernel(user_ids, movie_ids, user_table, movie_table, W, b):
    uid = user_ids.astype(jnp.int32)
    mid = movie_ids.astype(jnp.int32)
    wflat = W.reshape(-1).astype(jnp.float32)
    wu_col = wflat[:EMBED].reshape(1, EMBED)
    wm_col = wflat[EMBED:].reshape(1, EMBED)
    bvec = jnp.broadcast_to(b.astype(jnp.float32), (EMBED,))
    su, sm = _matvec2(user_table.T, movie_table.T, wu_col, wm_col)
    return _sc_combine(uid, mid, su, sm, bvec)


# speedup vs baseline: 1.0761x; 1.0482x over previous
import functools

import jax
import jax.numpy as jnp
from jax import lax
from jax.experimental import pallas as pl
from jax.experimental.pallas import tpu as pltpu
from jax.experimental.pallas import tpu_sc as plsc

BATCH = 16384
EMBED = 16
NUM_CORES = 2
NUM_SUBCORES = 16
NW = NUM_CORES * NUM_SUBCORES
B_PER_W = BATCH // NW
CHUNK = 128
N_CHUNKS = B_PER_W // CHUNK

MAX_RATING = 5.0
MIN_RATING = 0.5


U_BLK = 102400
M_BLK = 10240


def _matvec_body(ut_ref, mt_ref, w_ref, b_ref, ou_ref, om_ref):
    w = w_ref[...]
    dn = (((1,), (0,)), ((), ()))
    ou = jax.lax.dot_general(w[:, :EMBED], ut_ref[...], dn,
                             preferred_element_type=jnp.float32)
    om = jax.lax.dot_general(w[:, EMBED:], mt_ref[...], dn,
                             preferred_element_type=jnp.float32)
    ou_ref[...] = ou.reshape(ou_ref.shape) + b_ref[0, 0]
    om_ref[...] = om.reshape(om_ref.shape)


def _matvec2(ut, mt, w, b11):
    nu = ut.shape[1]
    nm = mt.shape[1]
    grid = (nu + U_BLK - 1) // U_BLK
    assert (nm + M_BLK - 1) // M_BLK == grid
    return pl.pallas_call(
        _matvec_body,
        grid=(grid,),
        in_specs=[
            pl.BlockSpec((EMBED, U_BLK), lambda i: (0, i)),
            pl.BlockSpec((EMBED, M_BLK), lambda i: (0, i)),
            pl.BlockSpec((1, 2 * EMBED), lambda i: (0, 0)),
            pl.BlockSpec((1, 1), lambda i: (0, 0)),
        ],
        out_specs=(
            pl.BlockSpec((U_BLK,), lambda i: (i,)),
            pl.BlockSpec((M_BLK,), lambda i: (i,)),
        ),
        out_shape=(
            jax.ShapeDtypeStruct((nu,), jnp.float32),
            jax.ShapeDtypeStruct((nm,), jnp.float32),
        ),
        compiler_params=pltpu.CompilerParams(
            dimension_semantics=("parallel",),
        ),
    )(ut, mt, w, b11)


def _sc_body(uid_hbm, mid_hbm, su_hbm, sm_hbm, out_hbm,
             uidx, midx, sus, sms, outv, usem, msem):
    wid = lax.axis_index("s") * NUM_CORES + lax.axis_index("c")
    base = wid * B_PER_W

    pltpu.sync_copy(uid_hbm.at[pl.ds(base, B_PER_W)], uidx)
    pltpu.sync_copy(mid_hbm.at[pl.ds(base, B_PER_W)], midx)

    copies = []
    for c in range(N_CHUNKS):
        sl = pl.ds(c * CHUNK, CHUNK)
        copies.append(pltpu.async_copy(su_hbm.at[uidx.at[sl]], sus.at[sl], usem))
        copies.append(pltpu.async_copy(sm_hbm.at[midx.at[sl]], sms.at[sl], msem))

    for cp in copies:
        cp.wait()

    def grp_body(g, carry):
        off = pl.multiple_of(g * EMBED, EMBED)
        v = sus[pl.ds(off, EMBED)] + sms[pl.ds(off, EMBED)]
        y = (MAX_RATING - MIN_RATING) / (1.0 + jnp.exp(-v)) + MIN_RATING
        outv[pl.ds(off, EMBED)] = y
        return carry

    lax.fori_loop(0, B_PER_W // EMBED, grp_body, 0)

    pltpu.sync_copy(outv, out_hbm.at[pl.ds(base, B_PER_W)])


@functools.partial(
    pl.kernel,
    mesh=plsc.VectorSubcoreMesh(core_axis_name="c", subcore_axis_name="s"),
    out_type=jax.ShapeDtypeStruct((BATCH,), jnp.float32),
    compiler_params=pltpu.CompilerParams(
        needs_layout_passes=False, use_tc_tiling_on_sc=False
    ),
    scratch_types=[
        pltpu.VMEM((B_PER_W,), jnp.int32),
        pltpu.VMEM((B_PER_W,), jnp.int32),
        pltpu.VMEM((B_PER_W,), jnp.float32),
        pltpu.VMEM((B_PER_W,), jnp.float32),
        pltpu.VMEM((B_PER_W,), jnp.float32),
        pltpu.SemaphoreType.DMA,
        pltpu.SemaphoreType.DMA,
    ],
)
def _sc_combine(*refs):
    _sc_body(*refs)


def kernel(user_ids, movie_ids, user_table, movie_table, W, b):
    uid = user_ids.astype(jnp.int32)
    mid = movie_ids.astype(jnp.int32)
    su, sm = _matvec2(
        user_table.T,
        movie_table.T,
        W.astype(jnp.float32),
        b.astype(jnp.float32).reshape(1, 1),
    )
    return _sc_combine(uid, mid, su, sm)


# speedup vs baseline: 1.0771x; 1.0009x over previous
import functools

import jax
import jax.numpy as jnp
from jax import lax
from jax.experimental import pallas as pl
from jax.experimental.pallas import tpu as pltpu
from jax.experimental.pallas import tpu_sc as plsc

BATCH = 16384
EMBED = 16
NUM_CORES = 2
NUM_SUBCORES = 16
NW = NUM_CORES * NUM_SUBCORES
B_PER_W = BATCH // NW
CHUNK = 128
N_CHUNKS = B_PER_W // CHUNK

MAX_RATING = 5.0
MIN_RATING = 0.5


U_BLK = 102400
M_BLK = 10240


def _matvec_body(ut_ref, mt_ref, w_ref, b_ref, ou_ref, om_ref):
    w = w_ref[...]
    dn = (((1,), (0,)), ((), ()))
    ou = jax.lax.dot_general(w[:, :EMBED], ut_ref[...], dn,
                             preferred_element_type=jnp.float32)
    om = jax.lax.dot_general(w[:, EMBED:], mt_ref[...], dn,
                             preferred_element_type=jnp.float32)
    ou_ref[...] = ou.reshape(ou_ref.shape) + b_ref[0, 0]
    om_ref[...] = om.reshape(om_ref.shape)


def _matvec2(ut, mt, w, b11):
    nu = ut.shape[1]
    nm = mt.shape[1]
    grid = (nu + U_BLK - 1) // U_BLK
    assert (nm + M_BLK - 1) // M_BLK == grid
    return pl.pallas_call(
        _matvec_body,
        grid=(grid,),
        in_specs=[
            pl.BlockSpec((EMBED, U_BLK), lambda i: (0, i)),
            pl.BlockSpec((EMBED, M_BLK), lambda i: (0, i)),
            pl.BlockSpec((1, 2 * EMBED), lambda i: (0, 0)),
            pl.BlockSpec((1, 1), lambda i: (0, 0)),
        ],
        out_specs=(
            pl.BlockSpec((U_BLK,), lambda i: (i,)),
            pl.BlockSpec((M_BLK,), lambda i: (i,)),
        ),
        out_shape=(
            jax.ShapeDtypeStruct((nu,), jnp.float32),
            jax.ShapeDtypeStruct((nm,), jnp.float32),
        ),
        compiler_params=pltpu.CompilerParams(
            dimension_semantics=("parallel",),
        ),
    )(ut, mt, w, b11)


def _sc_body(uid_hbm, mid_hbm, su_hbm, sm_hbm, out_hbm,
             uidx, midx, sus, sms, outv, usem, msem):
    wid = lax.axis_index("s") * NUM_CORES + lax.axis_index("c")
    base = wid * B_PER_W

    idc_u = pltpu.async_copy(uid_hbm.at[pl.ds(base, B_PER_W)], uidx, usem)
    idc_m = pltpu.async_copy(mid_hbm.at[pl.ds(base, B_PER_W)], midx, msem)
    idc_u.wait()
    idc_m.wait()

    copies = []
    for c in range(N_CHUNKS):
        sl = pl.ds(c * CHUNK, CHUNK)
        copies.append(pltpu.async_copy(su_hbm.at[uidx.at[sl]], sus.at[sl], usem))
        copies.append(pltpu.async_copy(sm_hbm.at[midx.at[sl]], sms.at[sl], msem))

    for cp in copies:
        cp.wait()

    for g in range(B_PER_W // EMBED):
        off = g * EMBED
        v = sus[pl.ds(off, EMBED)] + sms[pl.ds(off, EMBED)]
        y = (MAX_RATING - MIN_RATING) / (1.0 + jnp.exp(-v)) + MIN_RATING
        outv[pl.ds(off, EMBED)] = y

    pltpu.sync_copy(outv, out_hbm.at[pl.ds(base, B_PER_W)])


@functools.partial(
    pl.kernel,
    mesh=plsc.VectorSubcoreMesh(core_axis_name="c", subcore_axis_name="s"),
    out_type=jax.ShapeDtypeStruct((BATCH,), jnp.float32),
    compiler_params=pltpu.CompilerParams(
        needs_layout_passes=False, use_tc_tiling_on_sc=False
    ),
    scratch_types=[
        pltpu.VMEM((B_PER_W,), jnp.int32),
        pltpu.VMEM((B_PER_W,), jnp.int32),
        pltpu.VMEM((B_PER_W,), jnp.float32),
        pltpu.VMEM((B_PER_W,), jnp.float32),
        pltpu.VMEM((B_PER_W,), jnp.float32),
        pltpu.SemaphoreType.DMA,
        pltpu.SemaphoreType.DMA,
    ],
)
def _sc_combine(*refs):
    _sc_body(*refs)


def kernel(user_ids, movie_ids, user_table, movie_table, W, b):
    uid = user_ids.astype(jnp.int32)
    mid = movie_ids.astype(jnp.int32)
    su, sm = _matvec2(
        user_table.T,
        movie_table.T,
        W.astype(jnp.float32),
        b.astype(jnp.float32).reshape(1, 1),
    )
    return _sc_combine(uid, mid, su, sm)


# speedup vs baseline: 1.0793x; 1.0020x over previous
import functools

import jax
import jax.numpy as jnp
from jax import lax
from jax.experimental import pallas as pl
from jax.experimental.pallas import tpu as pltpu
from jax.experimental.pallas import tpu_sc as plsc

BATCH = 16384
EMBED = 16
NUM_CORES = 2
NUM_SUBCORES = 16
NW = NUM_CORES * NUM_SUBCORES
B_PER_W = BATCH // NW
CHUNK = 128
N_CHUNKS = B_PER_W // CHUNK

MAX_RATING = 5.0
MIN_RATING = 0.5


U_BLK = 102400
M_BLK = 10240


def _matvec_body(ut_ref, mt_ref, w_ref, b_ref, ou_ref, om_ref):
    w = w_ref[...]
    dn = (((1,), (0,)), ((), ()))
    ou = jax.lax.dot_general(w[:, :EMBED], ut_ref[...], dn,
                             preferred_element_type=jnp.float32)
    om = jax.lax.dot_general(w[:, EMBED:], mt_ref[...], dn,
                             preferred_element_type=jnp.float32)
    ou_ref[...] = ou.reshape(ou_ref.shape) + b_ref[0, 0]
    om_ref[...] = om.reshape(om_ref.shape)


def _matvec2(ut, mt, w, b11):
    nu = ut.shape[1]
    nm = mt.shape[1]
    grid = (nu + U_BLK - 1) // U_BLK
    assert (nm + M_BLK - 1) // M_BLK == grid
    return pl.pallas_call(
        _matvec_body,
        grid=(grid,),
        in_specs=[
            pl.BlockSpec((EMBED, U_BLK), lambda i: (0, i)),
            pl.BlockSpec((EMBED, M_BLK), lambda i: (0, i)),
            pl.BlockSpec((1, 2 * EMBED), lambda i: (0, 0)),
            pl.BlockSpec((1, 1), lambda i: (0, 0)),
        ],
        out_specs=(
            pl.BlockSpec((U_BLK,), lambda i: (i,)),
            pl.BlockSpec((M_BLK,), lambda i: (i,)),
        ),
        out_shape=(
            jax.ShapeDtypeStruct((nu,), jnp.float32),
            jax.ShapeDtypeStruct((nm,), jnp.float32),
        ),
        compiler_params=pltpu.CompilerParams(
            dimension_semantics=("parallel",),
        ),
    )(ut, mt, w, b11)


def _sc_body(uid_hbm, mid_hbm, su_hbm, sm_hbm, out_hbm,
             uidx, midx, sus, sms, outv, usem, msem):
    wid = lax.axis_index("s") * NUM_CORES + lax.axis_index("c")
    base = wid * B_PER_W

    idc_u = pltpu.async_copy(uid_hbm.at[pl.ds(base, B_PER_W)], uidx, usem)
    idc_m = pltpu.async_copy(mid_hbm.at[pl.ds(base, B_PER_W)], midx, msem)
    idc_u.wait()
    idc_m.wait()

    ucs, mcs = [], []
    for c in range(N_CHUNKS):
        sl = pl.ds(c * CHUNK, CHUNK)
        ucs.append(pltpu.async_copy(su_hbm.at[uidx.at[sl]], sus.at[sl], usem))
        mcs.append(pltpu.async_copy(sm_hbm.at[midx.at[sl]], sms.at[sl], msem))

    for c in range(N_CHUNKS):
        ucs[c].wait()
        mcs[c].wait()
        for g in range(c * (CHUNK // EMBED), (c + 1) * (CHUNK // EMBED)):
            off = g * EMBED
            v = sus[pl.ds(off, EMBED)] + sms[pl.ds(off, EMBED)]
            y = (MAX_RATING - MIN_RATING) / (1.0 + jnp.exp(-v)) + MIN_RATING
            outv[pl.ds(off, EMBED)] = y

    pltpu.sync_copy(outv, out_hbm.at[pl.ds(base, B_PER_W)])


@functools.partial(
    pl.kernel,
    mesh=plsc.VectorSubcoreMesh(core_axis_name="c", subcore_axis_name="s"),
    out_type=jax.ShapeDtypeStruct((BATCH,), jnp.float32),
    compiler_params=pltpu.CompilerParams(
        needs_layout_passes=False, use_tc_tiling_on_sc=False
    ),
    scratch_types=[
        pltpu.VMEM((B_PER_W,), jnp.int32),
        pltpu.VMEM((B_PER_W,), jnp.int32),
        pltpu.VMEM((B_PER_W,), jnp.float32),
        pltpu.VMEM((B_PER_W,), jnp.float32),
        pltpu.VMEM((B_PER_W,), jnp.float32),
        pltpu.SemaphoreType.DMA,
        pltpu.SemaphoreType.DMA,
    ],
)
def _sc_combine(*refs):
    _sc_body(*refs)


def kernel(user_ids, movie_ids, user_table, movie_table, W, b):
    uid = user_ids.astype(jnp.int32)
    mid = movie_ids.astype(jnp.int32)
    su, sm = _matvec2(
        user_table.T,
        movie_table.T,
        W.astype(jnp.float32),
        b.astype(jnp.float32).reshape(1, 1),
    )
    return _sc_combine(uid, mid, su, sm)
